# baseline (speedup 1.0000x reference)
import functools

import jax
import jax.numpy as jnp
from jax import lax
from jax.experimental import pallas as pl
from jax.experimental.pallas import tpu as pltpu
from jax.experimental.pallas import tpu_sc as plsc

N, E, D, DE = 10000, 320000, 128, 16

NC, NS, L = 2, 16, 16
NW = NC * NS
E0 = 128000
E1 = E - E0
CG = 80
GNB = 4
CS = 40
ZB = 40
NZB = N // ZB
NBUF = 6


def _silu(v):
    return v * (1.0 / (1.0 + jnp.exp(-v)))


D2 = D // 2


def _unpack_bf16(v):
    lo = lax.bitcast_convert_type(v << 16, jnp.float32)
    hi = lax.bitcast_convert_type(v & jnp.int32(-65536), jnp.float32)
    return lo, hi


def _node_pre(x, w1a, w1b):
    BN = 1000

    def body(x_ref, wa_ref, wb_ref, xa_ref, xb_ref):
        xv = x_ref[...]
        xa_ref[...] = jnp.dot(xv, wa_ref[...], preferred_element_type=jnp.float32)
        xb_ref[...] = jnp.dot(xv, wb_ref[...], preferred_element_type=jnp.float32)

    return pl.pallas_call(
        body,
        grid=(N // BN,),
        in_specs=[
            pl.BlockSpec((BN, D), lambda i: (i, 0)),
            pl.BlockSpec((D, D), lambda i: (0, 0)),
            pl.BlockSpec((D, D), lambda i: (0, 0)),
        ],
        out_specs=[pl.BlockSpec((BN, D), lambda i: (i, 0))] * 2,
        out_shape=[jax.ShapeDtypeStruct((N, D), jnp.float32)] * 2,
    )(x, w1a, w1b)


def _gather_sum(xa, xb, src, dst):
    ne = src.shape[0]
    ew = ne // NW
    nchunk = ew // CG
    mesh = plsc.VectorSubcoreMesh(core_axis_name="c", subcore_axis_name="s")

    @functools.partial(
        pl.kernel,
        mesh=mesh,
        out_type=jax.ShapeDtypeStruct((ne, D2), jnp.int32),
        scratch_types=[
            [pltpu.VMEM((CG,), jnp.int32)] * GNB,
            [pltpu.VMEM((CG,), jnp.int32)] * GNB,
            [pltpu.VMEM((CG, D), jnp.float32)] * GNB,
            [pltpu.VMEM((CG, D), jnp.float32)] * GNB,
            [pltpu.VMEM((CG, D2), jnp.int32)] * GNB,
            [pltpu.SemaphoreType.DMA] * GNB,
            [pltpu.SemaphoreType.DMA] * GNB,
            [pltpu.SemaphoreType.DMA] * GNB,
            [pltpu.SemaphoreType.DMA] * GNB,
            [pltpu.SemaphoreType.DMA] * GNB,
        ],
    )
    def k(xa_hbm, xb_hbm, src_hbm, dst_hbm, out_hbm, sibuf, dibuf,
          abuf, bbuf, obuf, ssi, sdi, sga, sgb, sout):
        wid = lax.axis_index("s") * NC + lax.axis_index("c")
        base = wid * ew

        def issue_idx(j, b):
            pltpu.async_copy(src_hbm.at[pl.ds(base + j * CG, CG)], sibuf[b],
                             ssi[b])
            pltpu.async_copy(dst_hbm.at[pl.ds(base + j * CG, CG)], dibuf[b],
                             sdi[b])

        def wait_idx(b):
            pltpu.make_async_copy(src_hbm.at[pl.ds(base, CG)], sibuf[b],
                                  ssi[b]).wait()
            pltpu.make_async_copy(dst_hbm.at[pl.ds(base, CG)], dibuf[b],
                                  sdi[b]).wait()

        def issue_g(b):
            pltpu.async_copy(xa_hbm.at[sibuf[b]], abuf[b], sga[b])
            pltpu.async_copy(xb_hbm.at[dibuf[b]], bbuf[b], sgb[b])

        def wait_g(b):
            pltpu.make_async_copy(xa_hbm.at[sibuf[b]], abuf[b], sga[b]).wait()
            pltpu.make_async_copy(xb_hbm.at[dibuf[b]], bbuf[b], sgb[b]).wait()

        for b in range(GNB):
            issue_idx(b, b)
        for b in range(2):
            wait_idx(b)
            issue_g(b)

        def quad(q, carry):
            for b in range(GNB):
                j = q * GNB + b
                p2 = (b + 2) % GNB

                @pl.when(jnp.logical_and(j >= GNB, j < nchunk + GNB))
                def _():
                    pltpu.make_async_copy(
                        obuf[b], out_hbm.at[pl.ds(base, CG)], sout[b]).wait()

                @pl.when(j < nchunk)
                def _():
                    wait_g(b)

                    @plsc.parallel_loop(0, CG, unroll=4)
                    def add_row(r):
                        for kk in range(D2 // L):
                            sl_lo = pl.ds(kk * L, L)
                            sl_hi = pl.ds(D2 + kk * L, L)
                            u = lax.bitcast_convert_type(
                                abuf[b][r, sl_lo] + bbuf[b][r, sl_lo],
                                jnp.int32)
                            v = lax.bitcast_convert_type(
                                abuf[b][r, sl_hi] + bbuf[b][r, sl_hi],
                                jnp.int32)
                            u = (u + 0x7FFF + ((u >> 16) & 1)) >> 16
                            v = (v + 0x7FFF + ((v >> 16) & 1)) & jnp.int32(
                                -65536)
                            obuf[b][r, sl_lo] = (u & 0xFFFF) | v

                    pltpu.async_copy(obuf[b],
                                     out_hbm.at[pl.ds(base + j * CG, CG)],
                                     sout[b])

                @pl.when(j + GNB < nchunk)
                def _():
                    issue_idx(j + GNB, b)

                @pl.when(j + 2 < nchunk)
                def _():
                    wait_idx(p2)
                    issue_g(p2)

            return carry

        lax.fori_loop(0, (nchunk + 2 * GNB) // GNB, quad, 0)

    return k(xa, xb, src, dst)


def _edge_mlp(s, ea, w1c, b1, w2, b2):
    ne = s.shape[0]
    BE = 2000

    def body(s_ref, ea_ref, w1c_ref, b1_ref, w2_ref, b2_ref, m_ref):
        s_lo, s_hi = _unpack_bf16(s_ref[...])
        g = jnp.concatenate([s_lo, s_hi], axis=1)
        pre = (g
               + jnp.dot(ea_ref[...], w1c_ref[...],
                         preferred_element_type=jnp.float32)
               + b1_ref[...])
        h = _silu(pre)
        m_ref[...] = _silu(jnp.dot(h, w2_ref[...],
                                   preferred_element_type=jnp.float32)
                           + b2_ref[...])

    return pl.pallas_call(
        body,
        grid=(ne // BE,),
        in_specs=[
            pl.BlockSpec((BE, D2), lambda i: (i, 0)),
            pl.BlockSpec((BE, DE), lambda i: (i, 0)),
            pl.BlockSpec((DE, D), lambda i: (0, 0)),
            pl.BlockSpec((1, D), lambda i: (0, 0)),
            pl.BlockSpec((D, D), lambda i: (0, 0)),
            pl.BlockSpec((1, D), lambda i: (0, 0)),
        ],
        out_specs=pl.BlockSpec((BE, D), lambda i: (i, 0)),
        out_shape=jax.ShapeDtypeStruct((ne, D), jnp.float32),
    )(s, ea, w1c, b1.reshape(1, D), w2, b2.reshape(1, D))


def _scatter_add(m, dst):
    ne = dst.shape[0]
    ew = ne // NW
    nchunks = ew // CS
    mesh = plsc.VectorSubcoreMesh(core_axis_name="c", subcore_axis_name="s")

    @functools.partial(
        pl.kernel,
        mesh=mesh,
        out_type=jax.ShapeDtypeStruct((NC, N, D), jnp.float32),
        scratch_types=[
            [pltpu.VMEM((CS,), jnp.int32)] * NBUF,
            [pltpu.VMEM((CS, D), jnp.float32)] * NBUF,
            pltpu.VMEM((ZB, D), jnp.float32),
            pltpu.VMEM_SHARED((N, D), jnp.float32),
            [pltpu.SemaphoreType.DMA] * NBUF,
            [pltpu.SemaphoreType.DMA] * NBUF,
            [pltpu.SemaphoreType.DMA] * NBUF,
        ],
    )
    def k(m_hbm, dst_hbm, out_hbm, dibuf, mbuf, zbuf, agg_sh, sidx, sld, ssc):
        cid = lax.axis_index("c")
        sid = lax.axis_index("s")
        wid = sid * NC + cid
        base = wid * ew

        def zrow(r, carry):
            for kk in range(D // L):
                zbuf[r, pl.ds(kk * L, L)] = jnp.zeros((L,), jnp.float32)
            return carry

        lax.fori_loop(0, ZB, zrow, 0)

        def zcp(t, carry):
            blk = sid + NS * t

            @pl.when(blk < NZB)
            def _():
                pltpu.sync_copy(zbuf, agg_sh.at[pl.ds(blk * ZB, ZB)])

            return carry

        lax.fori_loop(0, (NZB + NS - 1) // NS, zcp, 0)
        plsc.subcore_barrier()

        def issue_ld(j, b):
            pltpu.async_copy(m_hbm.at[pl.ds(base + j * CS, CS)], mbuf[b],
                             sld[b])
            pltpu.async_copy(dst_hbm.at[pl.ds(base + j * CS, CS)], dibuf[b],
                             sidx[b])

        def wait_ld(b):
            pltpu.make_async_copy(m_hbm.at[pl.ds(base, CS)], mbuf[b],
                                  sld[b]).wait()
            pltpu.make_async_copy(dst_hbm.at[pl.ds(base, CS)], dibuf[b],
                                  sidx[b]).wait()

        for b in range(4):
            issue_ld(b, b)

        def quad(q, carry):
            for b in range(NBUF):
                j = q * NBUF + b
                p4 = (b + 4) % NBUF

                @pl.when(j < nchunks)
                def _():
                    wait_ld(b)
                    pltpu.async_copy(mbuf[b], agg_sh.at[dibuf[b]], ssc[b],
                                     add=True)

                @pl.when(jnp.logical_and(j >= 2, j <= nchunks + 1))
                def _():
                    pltpu.make_async_copy(
                        mbuf[p4], agg_sh.at[dibuf[p4]], ssc[p4]).wait()

                @pl.when(j + 4 < nchunks)
                def _():
                    issue_ld(j + 4, p4)

            return carry

        lax.fori_loop(0, (nchunks + NBUF) // NBUF + 1, quad, 0)
        plsc.subcore_barrier()

        def wcp(t, carry):
            blk = sid + NS * t

            @pl.when(blk < NZB)
            def _():
                pltpu.sync_copy(agg_sh.at[pl.ds(blk * ZB, ZB)],
                                out_hbm.at[cid, pl.ds(blk * ZB, ZB)])

            return carry

        lax.fori_loop(0, (NZB + NS - 1) // NS, wcp, 0)

    return k(m, dst)


def _node_post(x, aggs, w3a, w3b, b3):
    BN = 1000
    NP = aggs.shape[0]

    def body(x_ref, a_ref, wa_ref, wb_ref, b3_ref, o_ref):
        a = a_ref[0]
        for i in range(1, NP):
            a = a + a_ref[i]
        acc = (jnp.dot(x_ref[...], wa_ref[...],
                       preferred_element_type=jnp.float32)
               + jnp.dot(a, wb_ref[...], preferred_element_type=jnp.float32)
               + b3_ref[...])
        o_ref[...] = _silu(acc)

    return pl.pallas_call(
        body,
        grid=(N // BN,),
        in_specs=[
            pl.BlockSpec((BN, D), lambda i: (i, 0)),
            pl.BlockSpec((NP, BN, D), lambda i: (0, i, 0)),
            pl.BlockSpec((D, D), lambda i: (0, 0)),
            pl.BlockSpec((D, D), lambda i: (0, 0)),
            pl.BlockSpec((1, D), lambda i: (0, 0)),
        ],
        out_specs=pl.BlockSpec((BN, D), lambda i: (i, 0)),
        out_shape=jax.ShapeDtypeStruct((N, D), jnp.float32),
    )(x, aggs, w3a, w3b, b3.reshape(1, D))


def kernel(x, edge_index, edge_attr, W1, b1, W2, b2, W3, b3):
    src = edge_index[0]
    dst = edge_index[1]
    xa, xb = _node_pre(x, W1[:D], W1[D:2 * D])
    w1c = W1[2 * D:]

    s0 = _gather_sum(xa, xb, src[:E0], dst[:E0])
    s1 = _gather_sum(xa, xb, src[E0:], dst[E0:])
    m0 = _edge_mlp(s0, edge_attr[:E0], w1c, b1, W2, b2)
    agg0 = _scatter_add(m0, dst[:E0])
    m1 = _edge_mlp(s1, edge_attr[E0:], w1c, b1, W2, b2)
    agg1 = _scatter_add(m1, dst[E0:])

    aggs = jnp.concatenate([agg0, agg1], axis=0)
    return _node_post(x, aggs, W3[:D], W3[D:], b3)


# speedup vs baseline: 1.0142x; 1.0142x over previous
import functools

import jax
import jax.numpy as jnp
from jax import lax
from jax.experimental import pallas as pl
from jax.experimental.pallas import tpu as pltpu
from jax.experimental.pallas import tpu_sc as plsc

N, E, D, DE = 10000, 320000, 128, 16

NC, NS, L = 2, 16, 16
NW = NC * NS
E0 = E // 2
CG = 40
GNB = 6
CS = 40
ZB = 40
NZB = N // ZB
NBUF = 6


def _silu(v):
    return v * (1.0 / (1.0 + jnp.exp(-v)))


D2 = D // 2


def _unpack_bf16(v):
    lo = lax.bitcast_convert_type(v << 16, jnp.float32)
    hi = lax.bitcast_convert_type(v & jnp.int32(-65536), jnp.float32)
    return lo, hi


def _node_pre(x, w1a, w1b):
    BN = 1000

    def body(x_ref, wa_ref, wb_ref, xa_ref, xb_ref):
        xv = x_ref[...]
        xa_ref[...] = jnp.dot(xv, wa_ref[...], preferred_element_type=jnp.float32)
        xb_ref[...] = jnp.dot(xv, wb_ref[...], preferred_element_type=jnp.float32)

    return pl.pallas_call(
        body,
        grid=(N // BN,),
        in_specs=[
            pl.BlockSpec((BN, D), lambda i: (i, 0)),
            pl.BlockSpec((D, D), lambda i: (0, 0)),
            pl.BlockSpec((D, D), lambda i: (0, 0)),
        ],
        out_specs=[pl.BlockSpec((BN, D), lambda i: (i, 0))] * 2,
        out_shape=[jax.ShapeDtypeStruct((N, D), jnp.float32)] * 2,
    )(x, w1a, w1b)


def _gather_sum(xa, xb, src, dst):
    ne = src.shape[0]
    ew = ne // NW
    nchunk = ew // CG
    mesh = plsc.VectorSubcoreMesh(core_axis_name="c", subcore_axis_name="s")

    @functools.partial(
        pl.kernel,
        mesh=mesh,
        out_type=jax.ShapeDtypeStruct((ne, D2), jnp.int32),
        scratch_types=[
            [pltpu.VMEM((CG,), jnp.int32)] * GNB,
            [pltpu.VMEM((CG,), jnp.int32)] * GNB,
            [pltpu.VMEM((CG, D), jnp.float32)] * GNB,
            [pltpu.VMEM((CG, D), jnp.float32)] * GNB,
            [pltpu.VMEM((CG, D2), jnp.int32)] * GNB,
            [pltpu.SemaphoreType.DMA] * GNB,
            [pltpu.SemaphoreType.DMA] * GNB,
            [pltpu.SemaphoreType.DMA] * GNB,
            [pltpu.SemaphoreType.DMA] * GNB,
            [pltpu.SemaphoreType.DMA] * GNB,
        ],
    )
    def k(xa_hbm, xb_hbm, src_hbm, dst_hbm, out_hbm, sibuf, dibuf,
          abuf, bbuf, obuf, ssi, sdi, sga, sgb, sout):
        wid = lax.axis_index("s") * NC + lax.axis_index("c")
        base = wid * ew

        def issue_idx(j, b):
            pltpu.async_copy(src_hbm.at[pl.ds(base + j * CG, CG)], sibuf[b],
                             ssi[b])
            pltpu.async_copy(dst_hbm.at[pl.ds(base + j * CG, CG)], dibuf[b],
                             sdi[b])

        def wait_idx(b):
            pltpu.make_async_copy(src_hbm.at[pl.ds(base, CG)], sibuf[b],
                                  ssi[b]).wait()
            pltpu.make_async_copy(dst_hbm.at[pl.ds(base, CG)], dibuf[b],
                                  sdi[b]).wait()

        def issue_g(b):
            pltpu.async_copy(xa_hbm.at[sibuf[b]], abuf[b], sga[b])
            pltpu.async_copy(xb_hbm.at[dibuf[b]], bbuf[b], sgb[b])

        def wait_g(b):
            pltpu.make_async_copy(xa_hbm.at[sibuf[b]], abuf[b], sga[b]).wait()
            pltpu.make_async_copy(xb_hbm.at[dibuf[b]], bbuf[b], sgb[b]).wait()

        for b in range(GNB):
            issue_idx(b, b)
        for b in range(4):
            wait_idx(b)
            issue_g(b)

        def quad(q, carry):
            for b in range(GNB):
                j = q * GNB + b
                p4 = (b + 4) % GNB

                @pl.when(jnp.logical_and(j >= GNB, j < nchunk + GNB))
                def _():
                    pltpu.make_async_copy(
                        obuf[b], out_hbm.at[pl.ds(base, CG)], sout[b]).wait()

                @pl.when(j < nchunk)
                def _():
                    wait_g(b)

                    @plsc.parallel_loop(0, CG, unroll=4)
                    def add_row(r):
                        for kk in range(D2 // L):
                            sl_lo = pl.ds(kk * L, L)
                            sl_hi = pl.ds(D2 + kk * L, L)
                            u = lax.bitcast_convert_type(
                                abuf[b][r, sl_lo] + bbuf[b][r, sl_lo],
                                jnp.int32)
                            v = lax.bitcast_convert_type(
                                abuf[b][r, sl_hi] + bbuf[b][r, sl_hi],
                                jnp.int32)
                            u = (u + 0x7FFF + ((u >> 16) & 1)) >> 16
                            v = (v + 0x7FFF + ((v >> 16) & 1)) & jnp.int32(
                                -65536)
                            obuf[b][r, sl_lo] = (u & 0xFFFF) | v

                    pltpu.async_copy(obuf[b],
                                     out_hbm.at[pl.ds(base + j * CG, CG)],
                                     sout[b])

                @pl.when(j + GNB < nchunk)
                def _():
                    issue_idx(j + GNB, b)

                @pl.when(j + 4 < nchunk)
                def _():
                    wait_idx(p4)
                    issue_g(p4)

            return carry

        lax.fori_loop(0, (nchunk + 2 * GNB) // GNB, quad, 0)

    return k(xa, xb, src, dst)


def _edge_mlp(s, ea, w1c, b1, w2, b2):
    ne = s.shape[0]
    BE = 2000

    def body(s_ref, ea_ref, w1c_ref, b1_ref, w2_ref, b2_ref, m_ref):
        s_lo, s_hi = _unpack_bf16(s_ref[...])
        g = jnp.concatenate([s_lo, s_hi], axis=1)
        pre = (g
               + jnp.dot(ea_ref[...], w1c_ref[...],
                         preferred_element_type=jnp.float32)
               + b1_ref[...])
        h = _silu(pre)
        m_ref[...] = _silu(jnp.dot(h, w2_ref[...],
                                   preferred_element_type=jnp.float32)
                           + b2_ref[...])

    return pl.pallas_call(
        body,
        grid=(ne // BE,),
        in_specs=[
            pl.BlockSpec((BE, D2), lambda i: (i, 0)),
            pl.BlockSpec((BE, DE), lambda i: (i, 0)),
            pl.BlockSpec((DE, D), lambda i: (0, 0)),
            pl.BlockSpec((1, D), lambda i: (0, 0)),
            pl.BlockSpec((D, D), lambda i: (0, 0)),
            pl.BlockSpec((1, D), lambda i: (0, 0)),
        ],
        out_specs=pl.BlockSpec((BE, D), lambda i: (i, 0)),
        out_shape=jax.ShapeDtypeStruct((ne, D), jnp.float32),
    )(s, ea, w1c, b1.reshape(1, D), w2, b2.reshape(1, D))


def _scatter_add(m, dst):
    ne = dst.shape[0]
    ew = ne // NW
    nchunks = ew // CS
    mesh = plsc.VectorSubcoreMesh(core_axis_name="c", subcore_axis_name="s")

    @functools.partial(
        pl.kernel,
        mesh=mesh,
        out_type=jax.ShapeDtypeStruct((NC, N, D), jnp.float32),
        scratch_types=[
            [pltpu.VMEM((CS,), jnp.int32)] * NBUF,
            [pltpu.VMEM((CS, D), jnp.float32)] * NBUF,
            pltpu.VMEM((ZB, D), jnp.float32),
            pltpu.VMEM_SHARED((N, D), jnp.float32),
            [pltpu.SemaphoreType.DMA] * NBUF,
            [pltpu.SemaphoreType.DMA] * NBUF,
            [pltpu.SemaphoreType.DMA] * NBUF,
        ],
    )
    def k(m_hbm, dst_hbm, out_hbm, dibuf, mbuf, zbuf, agg_sh, sidx, sld, ssc):
        cid = lax.axis_index("c")
        sid = lax.axis_index("s")
        wid = sid * NC + cid
        base = wid * ew

        def zrow(r, carry):
            for kk in range(D // L):
                zbuf[r, pl.ds(kk * L, L)] = jnp.zeros((L,), jnp.float32)
            return carry

        lax.fori_loop(0, ZB, zrow, 0)

        def zcp(t, carry):
            blk = sid + NS * t

            @pl.when(blk < NZB)
            def _():
                pltpu.sync_copy(zbuf, agg_sh.at[pl.ds(blk * ZB, ZB)])

            return carry

        lax.fori_loop(0, (NZB + NS - 1) // NS, zcp, 0)
        plsc.subcore_barrier()

        def issue_ld(j, b):
            pltpu.async_copy(m_hbm.at[pl.ds(base + j * CS, CS)], mbuf[b],
                             sld[b])
            pltpu.async_copy(dst_hbm.at[pl.ds(base + j * CS, CS)], dibuf[b],
                             sidx[b])

        def wait_ld(b):
            pltpu.make_async_copy(m_hbm.at[pl.ds(base, CS)], mbuf[b],
                                  sld[b]).wait()
            pltpu.make_async_copy(dst_hbm.at[pl.ds(base, CS)], dibuf[b],
                                  sidx[b]).wait()

        for b in range(4):
            issue_ld(b, b)

        def quad(q, carry):
            for b in range(NBUF):
                j = q * NBUF + b
                p4 = (b + 4) % NBUF

                @pl.when(j < nchunks)
                def _():
                    wait_ld(b)
                    pltpu.async_copy(mbuf[b], agg_sh.at[dibuf[b]], ssc[b],
                                     add=True)

                @pl.when(jnp.logical_and(j >= 2, j <= nchunks + 1))
                def _():
                    pltpu.make_async_copy(
                        mbuf[p4], agg_sh.at[dibuf[p4]], ssc[p4]).wait()

                @pl.when(j + 4 < nchunks)
                def _():
                    issue_ld(j + 4, p4)

            return carry

        lax.fori_loop(0, (nchunks + NBUF) // NBUF + 1, quad, 0)
        plsc.subcore_barrier()

        def wcp(t, carry):
            blk = sid + NS * t

            @pl.when(blk < NZB)
            def _():
                pltpu.sync_copy(agg_sh.at[pl.ds(blk * ZB, ZB)],
                                out_hbm.at[cid, pl.ds(blk * ZB, ZB)])

            return carry

        lax.fori_loop(0, (NZB + NS - 1) // NS, wcp, 0)

    return k(m, dst)


def _node_post(x, aggs, w3a, w3b, b3):
    BN = 1000
    NP = aggs.shape[0]

    def body(x_ref, a_ref, wa_ref, wb_ref, b3_ref, o_ref):
        a = a_ref[0]
        for i in range(1, NP):
            a = a + a_ref[i]
        acc = (jnp.dot(x_ref[...], wa_ref[...],
                       preferred_element_type=jnp.float32)
               + jnp.dot(a, wb_ref[...], preferred_element_type=jnp.float32)
               + b3_ref[...])
        o_ref[...] = _silu(acc)

    return pl.pallas_call(
        body,
        grid=(N // BN,),
        in_specs=[
            pl.BlockSpec((BN, D), lambda i: (i, 0)),
            pl.BlockSpec((NP, BN, D), lambda i: (0, i, 0)),
            pl.BlockSpec((D, D), lambda i: (0, 0)),
            pl.BlockSpec((D, D), lambda i: (0, 0)),
            pl.BlockSpec((1, D), lambda i: (0, 0)),
        ],
        out_specs=pl.BlockSpec((BN, D), lambda i: (i, 0)),
        out_shape=jax.ShapeDtypeStruct((N, D), jnp.float32),
    )(x, aggs, w3a, w3b, b3.reshape(1, D))


def kernel(x, edge_index, edge_attr, W1, b1, W2, b2, W3, b3):
    src = edge_index[0]
    dst = edge_index[1]
    xa, xb = _node_pre(x, W1[:D], W1[D:2 * D])
    w1c = W1[2 * D:]

    s0 = _gather_sum(xa, xb, src[:E0], dst[:E0])
    s1 = _gather_sum(xa, xb, src[E0:], dst[E0:])
    m0 = _edge_mlp(s0, edge_attr[:E0], w1c, b1, W2, b2)
    agg0 = _scatter_add(m0, dst[:E0])
    m1 = _edge_mlp(s1, edge_attr[E0:], w1c, b1, W2, b2)
    agg1 = _scatter_add(m1, dst[E0:])

    aggs = jnp.concatenate([agg0, agg1], axis=0)
    return _node_post(x, aggs, W3[:D], W3[D:], b3)
